# baseline (speedup 1.0000x reference)
import functools

import jax
import jax.numpy as jnp
from jax import lax
from jax.experimental import pallas as pl
from jax.experimental.pallas import tpu as pltpu
from jax.experimental.pallas import tpu_sc as plsc

_PAD_IDX = 1
_NEG = -1e18


_SC_CHUNK = 80


@functools.lru_cache(maxsize=None)
def _make_sc_gather(n_idx: int, table_len: int):
    info = plsc.get_sparse_core_info()
    n_workers = info.num_cores * info.num_subcores
    per = n_idx // n_workers
    assert per * n_workers == n_idx and per % _SC_CHUNK == 0
    n_chunks = per // _SC_CHUNK
    mesh = plsc.VectorSubcoreMesh(core_axis_name="c", subcore_axis_name="s")

    @functools.partial(
        pl.kernel,
        out_type=jax.ShapeDtypeStruct((n_workers, n_chunks, _SC_CHUNK),
                                      jnp.float32),
        mesh=mesh,
        scratch_types=[
            pltpu.VMEM((n_chunks, _SC_CHUNK), jnp.int32),
            pltpu.VMEM((n_chunks, _SC_CHUNK), jnp.float32),
            pltpu.SemaphoreType.DMA,
        ],
    )
    def gather_kernel(table_hbm, idx_hbm, out_hbm, idx_v, rows_v, sem):
        wid = lax.axis_index("s") * info.num_cores + lax.axis_index("c")
        pltpu.sync_copy(idx_hbm.at[wid], idx_v)
        copies = [
            pltpu.async_copy(table_hbm.at[idx_v.at[j]], rows_v.at[j], sem)
            for j in range(n_chunks)
        ]
        for c in copies:
            c.wait()
        pltpu.sync_copy(rows_v, out_hbm.at[wid])

    def run(table_flat, idx_flat):
        idx3 = idx_flat.reshape(n_workers, n_chunks, _SC_CHUNK)
        return gather_kernel(table_flat, idx3).reshape(-1)

    return run


def _pool_body(ns_ref, node_ref, ew_ref, nbr_ref, ir_ref, w_ref, b_ref,
               out_ref):
    prod = ew_ref[...][..., None] * nbr_ref[...]
    masked = jnp.where(prod == 0.0, _NEG, prod)
    m = jnp.max(masked, axis=2)
    ir = jnp.where(ns_ref[...] == _PAD_IDX, 1.0,
                   ir_ref[...])[..., None]
    emb = (1.0 - ir) * m + ir * node_ref[...]
    s = jnp.sum(emb, axis=1)
    x = lax.dot_general(s, w_ref[...], (((1,), (1,)), ((), ())),
                        preferred_element_type=jnp.float32)
    x = jnp.maximum(x + b_ref[...], 0.0)
    x = x - jnp.max(x, axis=1, keepdims=True)
    e = jnp.exp(x)
    out_ref[...] = e / jnp.sum(e, axis=1, keepdims=True)


def _pool_call(ns, node, ew, nbr, ir, w, b2, block_b: int):
    batch, seq_len, k_nbrs, dim = nbr.shape
    out_dim = w.shape[0]
    grid = (batch // block_b,)
    return pl.pallas_call(
        _pool_body,
        grid=grid,
        in_specs=[
            pl.BlockSpec((block_b, seq_len), lambda i: (i, 0)),
            pl.BlockSpec((block_b, seq_len, dim), lambda i: (i, 0, 0)),
            pl.BlockSpec((block_b, seq_len, k_nbrs), lambda i: (i, 0, 0)),
            pl.BlockSpec((block_b, seq_len, k_nbrs, dim),
                         lambda i: (i, 0, 0, 0)),
            pl.BlockSpec((block_b, seq_len), lambda i: (i, 0)),
            pl.BlockSpec((out_dim, dim), lambda i: (0, 0)),
            pl.BlockSpec((1, out_dim), lambda i: (0, 0)),
        ],
        out_specs=pl.BlockSpec((block_b, out_dim), lambda i: (i, 0)),
        out_shape=jax.ShapeDtypeStruct((batch, out_dim), jnp.float32),
    )(ns, node, ew, nbr, ir, w, b2)


def kernel(node_sets, embedded_node, edge_weight, embedded_neighbor_node,
           information_rate, W, b):
    batch, seq_len = node_sets.shape
    ns = node_sets.astype(jnp.int32)
    table = information_rate.reshape(-1)
    ir = _make_sc_gather(batch * seq_len, table.shape[0])(
        table, ns.reshape(-1))
    ir = ir.reshape(batch, seq_len)
    return _pool_call(ns, embedded_node, edge_weight, embedded_neighbor_node,
                      ir, W, b.reshape(1, -1), block_b=64)


# speedup vs baseline: 1.0039x; 1.0039x over previous
import functools

import jax
import jax.numpy as jnp
from jax import lax
from jax.experimental import pallas as pl
from jax.experimental.pallas import tpu as pltpu
from jax.experimental.pallas import tpu_sc as plsc

_PAD_IDX = 1
_NEG = -1e18


_SC_CHUNK = 80


@functools.lru_cache(maxsize=None)
def _make_sc_gather(n_idx: int, table_len: int):
    info = plsc.get_sparse_core_info()
    n_workers = info.num_cores * info.num_subcores
    per = n_idx // n_workers
    assert per * n_workers == n_idx and per % _SC_CHUNK == 0
    n_chunks = per // _SC_CHUNK
    mesh = plsc.VectorSubcoreMesh(core_axis_name="c", subcore_axis_name="s")

    @functools.partial(
        pl.kernel,
        out_type=jax.ShapeDtypeStruct((n_workers, n_chunks, _SC_CHUNK),
                                      jnp.float32),
        mesh=mesh,
        scratch_types=[
            pltpu.VMEM((n_chunks, _SC_CHUNK), jnp.int32),
            pltpu.VMEM((n_chunks, _SC_CHUNK), jnp.float32),
            pltpu.SemaphoreType.DMA,
        ],
    )
    def gather_kernel(table_hbm, idx_hbm, out_hbm, idx_v, rows_v, sem):
        wid = lax.axis_index("s") * info.num_cores + lax.axis_index("c")
        pltpu.sync_copy(idx_hbm.at[wid], idx_v)
        copies = [
            pltpu.async_copy(table_hbm.at[idx_v.at[j]], rows_v.at[j], sem)
            for j in range(n_chunks)
        ]
        for c in copies:
            c.wait()
        pltpu.sync_copy(rows_v, out_hbm.at[wid])

    def run(table_flat, idx_flat):
        idx3 = idx_flat.reshape(n_workers, n_chunks, _SC_CHUNK)
        return gather_kernel(table_flat, idx3).reshape(-1)

    return run


def _pool_body(ns_ref, node_ref, ew_ref, nbr_ref, ir_ref, w_ref, b_ref,
               out_ref):
    prod = ew_ref[...][..., None] * nbr_ref[...]
    masked = jnp.where(prod == 0.0, _NEG, prod)
    m = jnp.max(masked, axis=2)
    ir = jnp.where(ns_ref[...] == _PAD_IDX, 1.0,
                   ir_ref[...])[..., None]
    emb = (1.0 - ir) * m + ir * node_ref[...]
    s = jnp.sum(emb, axis=1)
    x = lax.dot_general(s, w_ref[...], (((1,), (1,)), ((), ())),
                        preferred_element_type=jnp.float32)
    x = jnp.maximum(x + b_ref[...], 0.0)
    x = x - jnp.max(x, axis=1, keepdims=True)
    e = jnp.exp(x)
    out_ref[...] = e / jnp.sum(e, axis=1, keepdims=True)


def _pool_call(ns, node, ew, nbr, ir, w, b2, block_b: int):
    batch, seq_len, k_nbrs, dim = nbr.shape
    out_dim = w.shape[0]
    grid = (batch // block_b,)
    return pl.pallas_call(
        _pool_body,
        grid=grid,
        in_specs=[
            pl.BlockSpec((block_b, seq_len), lambda i: (i, 0)),
            pl.BlockSpec((block_b, seq_len, dim), lambda i: (i, 0, 0)),
            pl.BlockSpec((block_b, seq_len, k_nbrs), lambda i: (i, 0, 0)),
            pl.BlockSpec((block_b, seq_len, k_nbrs, dim),
                         lambda i: (i, 0, 0, 0)),
            pl.BlockSpec((block_b, seq_len), lambda i: (i, 0)),
            pl.BlockSpec((out_dim, dim), lambda i: (0, 0)),
            pl.BlockSpec((1, out_dim), lambda i: (0, 0)),
        ],
        out_specs=pl.BlockSpec((block_b, out_dim), lambda i: (i, 0)),
        out_shape=jax.ShapeDtypeStruct((batch, out_dim), jnp.float32),
    )(ns, node, ew, nbr, ir, w, b2)


def kernel(node_sets, embedded_node, edge_weight, embedded_neighbor_node,
           information_rate, W, b):
    batch, seq_len = node_sets.shape
    ns = node_sets.astype(jnp.int32)
    ir = jnp.take(information_rate.reshape(-1), ns.reshape(-1)).reshape(
        batch, seq_len)
    return _pool_call(ns, embedded_node, edge_weight, embedded_neighbor_node,
                      ir, W, b.reshape(1, -1), block_b=64)
